# Initial kernel scaffold; baseline (speedup 1.0000x reference)
import jax
import jax.numpy as jnp
from jax.experimental import pallas as pl


def kernel(atoms, bonds, tetras, cistrans, emb, marker, Wl, Wr, a, W1, b1, W2, b2, W3, b3):
    raise NotImplementedError("write your pallas kernel here")


# speedup vs baseline: 1.0026x; 1.0026x over previous
import numpy as np
import jax
import jax.numpy as jnp
from jax import lax
from jax.experimental import pallas as pl
from jax.experimental.pallas import tpu as pltpu

_N = 10000
_D = 256
_H = 8
_DH = 32
_F = 1024
_L = 3
_NORM = 0.0625

_BLK = 400


def _leaky(x):
    return jnp.where(x >= 0, x, 0.2 * x)


def _gelu(x):
    return 0.5 * x * (1.0 + lax.erf(x * np.float32(1.0 / np.sqrt(2.0))))


def _ng_body(h_ref, w_ref, o_ref):
    o_ref[...] = jnp.dot(h_ref[...], w_ref[...],
                         preferred_element_type=jnp.float32)


def _matmul_ng(h, Wlr):
    grid = _N // _BLK
    return pl.pallas_call(
        _ng_body,
        grid=(grid,),
        in_specs=[pl.BlockSpec((_BLK, _D), lambda i: (i, 0)),
                  pl.BlockSpec((_D, 2 * _D), lambda i: (0, 0))],
        out_specs=pl.BlockSpec((_BLK, 2 * _D), lambda i: (i, 0)),
        out_shape=jax.ShapeDtypeStruct((_N, 2 * _D), jnp.float32),
    )(h, Wlr)


def _ffn_body(x_ref, w1_ref, b1_ref, w2_ref, b2_ref, w3_ref, b3_ref, o_ref):
    z = _gelu(jnp.dot(x_ref[...], w1_ref[...],
                      preferred_element_type=jnp.float32) + b1_ref[...])
    z = _gelu(jnp.dot(z, w2_ref[...],
                      preferred_element_type=jnp.float32) + b2_ref[...])
    o_ref[...] = jnp.dot(z, w3_ref[...],
                         preferred_element_type=jnp.float32) + b3_ref[...]


def _ffn(x, W1, b1, W2, b2, W3, b3):
    grid = _N // _BLK
    return pl.pallas_call(
        _ffn_body,
        grid=(grid,),
        in_specs=[pl.BlockSpec((_BLK, 2 * _D), lambda i: (i, 0)),
                  pl.BlockSpec((2 * _D, _F), lambda i: (0, 0)),
                  pl.BlockSpec((1, _F), lambda i: (0, 0)),
                  pl.BlockSpec((_F, _F), lambda i: (0, 0)),
                  pl.BlockSpec((1, _F), lambda i: (0, 0)),
                  pl.BlockSpec((_F, _D), lambda i: (0, 0)),
                  pl.BlockSpec((1, _D), lambda i: (0, 0))],
        out_specs=pl.BlockSpec((_BLK, _D), lambda i: (i, 0)),
        out_shape=jax.ShapeDtypeStruct((_N, _D), jnp.float32),
    )(x, W1, b1.reshape(1, _F), W2, b2.reshape(1, _F), W3, b3.reshape(1, _D))


def kernel(atoms, bonds, tetras, cistrans, emb, marker, Wl, Wr, a, W1, b1,
           W2, b2, W3, b3):
    n = atoms.shape[0]
    h = emb[atoms] * _NORM
    idx = jnp.concatenate([bonds, bonds[:, ::-1]], axis=0)
    src = idx[:, 0]
    dst = idx[:, 1]
    for l in range(_L):
        Wlr = jnp.concatenate([Wl[l], Wr[l]], axis=1)
        ng = _matmul_ng(h, Wlr)
        ngl = ng[:, :_D].reshape(n, _H, _DH)
        ngr = ng[:, _D:].reshape(n, _H, _DH)
        gl = ngl[src]
        gr = ngr[dst]
        eps = jnp.einsum('ehd,d->eh', _leaky(gl + gr), a[l])
        smax = jax.ops.segment_max(eps, src, num_segments=n)[src]
        sexp = jnp.exp(eps - smax)
        ssum = jax.ops.segment_sum(sexp, src, num_segments=n)[src]
        alpha = sexp / ssum
        byedge = gr * alpha[:, :, None]
        agg = jax.ops.segment_sum(byedge, src, num_segments=n)
        cnt = jax.ops.segment_sum(jnp.ones_like(src), src, num_segments=n)
        attn = jnp.where((cnt > 0)[:, None, None], agg, byedge[:n]).reshape(n, -1)
        ff_in = jnp.concatenate([attn, h], axis=1)
        h = _ffn(ff_in, W1[l], b1[l], W2[l], b2[l], W3[l], b3[l])
    return h


# speedup vs baseline: 3.4905x; 3.4814x over previous
import functools
import numpy as np
import jax
import jax.numpy as jnp
from jax import lax
from jax.experimental import pallas as pl
from jax.experimental.pallas import tpu as pltpu
from jax.experimental.pallas import tpu_sc as plsc

_N = 10000
_NP = 10016
_D = 256
_H = 8
_DH = 32
_F = 1024
_L = 3
_NORM = 0.0625

_EE = 320000
_NWORK = 32
_CH = 10016
_EEP = _NWORK * _CH
_B = 32
_NBLK = _CH // _B

_BLK = 400


def _gelu(x):
    return 0.5 * x * (1.0 + lax.erf(x * np.float32(1.0 / np.sqrt(2.0))))


def _l0_body(atoms_ref, emb_ref, wlr_ref, h_ref, ngl_ref, ngr_ref):
    at = atoms_ref[0, 0, :]
    onehot = (at[:, None] == lax.iota(jnp.int32, 128)[None, :]).astype(jnp.float32)
    embn = emb_ref[...] * _NORM
    h = jnp.dot(onehot, embn, preferred_element_type=jnp.float32)
    h_ref[...] = h
    embw = jnp.dot(embn, wlr_ref[...], preferred_element_type=jnp.float32)
    ng = jnp.dot(onehot, embw, preferred_element_type=jnp.float32)
    ngl_ref[...] = ng[:, :_D]
    ngr_ref[...] = ng[:, _D:]


def _layer0(atoms, emb, Wlr):
    atoms3 = atoms.astype(jnp.int32).reshape(_N // _BLK, 1, _BLK)
    grid = _N // _BLK
    return pl.pallas_call(
        _l0_body,
        grid=(grid,),
        in_specs=[pl.BlockSpec((1, 1, _BLK), lambda i: (i, 0, 0)),
                  pl.BlockSpec((128, _D), lambda i: (0, 0)),
                  pl.BlockSpec((_D, 2 * _D), lambda i: (0, 0))],
        out_specs=[pl.BlockSpec((_BLK, _D), lambda i: (i, 0)),
                   pl.BlockSpec((_BLK, _D), lambda i: (i, 0)),
                   pl.BlockSpec((_BLK, _D), lambda i: (i, 0))],
        out_shape=[jax.ShapeDtypeStruct((_N, _D), jnp.float32),
                   jax.ShapeDtypeStruct((_N, _D), jnp.float32),
                   jax.ShapeDtypeStruct((_N, _D), jnp.float32)],
    )(atoms3, emb, Wlr)


def _ng_body(h_ref, wl_ref, wr_ref, ol_ref, or_ref):
    h = h_ref[...]
    ol_ref[...] = jnp.dot(h, wl_ref[...], preferred_element_type=jnp.float32)
    or_ref[...] = jnp.dot(h, wr_ref[...], preferred_element_type=jnp.float32)


def _matmul_ng(h, Wl, Wr):
    grid = _N // _BLK
    return pl.pallas_call(
        _ng_body,
        grid=(grid,),
        in_specs=[pl.BlockSpec((_BLK, _D), lambda i: (i, 0)),
                  pl.BlockSpec((_D, _D), lambda i: (0, 0)),
                  pl.BlockSpec((_D, _D), lambda i: (0, 0))],
        out_specs=[pl.BlockSpec((_BLK, _D), lambda i: (i, 0)),
                   pl.BlockSpec((_BLK, _D), lambda i: (i, 0))],
        out_shape=[jax.ShapeDtypeStruct((_N, _D), jnp.float32),
                   jax.ShapeDtypeStruct((_N, _D), jnp.float32)],
    )(h, Wl, Wr)


def _reduce_body(p_ref, o_ref):
    o_ref[...] = (p_ref[0] + p_ref[1]).reshape(_NP * 16 // 128, 128)


def _reduce_ssum(parts):
    return pl.pallas_call(
        _reduce_body,
        grid=(1,),
        in_specs=[pl.BlockSpec((2, _NP * 16), lambda i: (0, 0))],
        out_specs=pl.BlockSpec((_NP * 16 // 128, 128), lambda i: (0, 0)),
        out_shape=jax.ShapeDtypeStruct((_NP * 16 // 128, 128), jnp.float32),
    )(parts)


def _ffn_body(agg_ref, h_ref, w1_ref, b1_ref, w2_ref, b2_ref, w3_ref, b3_ref,
              o_ref):
    ag = agg_ref[...]
    attn = jnp.concatenate([ag[0, 0] + ag[1, 0], ag[0, 1] + ag[1, 1]], axis=1)
    x = jnp.concatenate([attn, h_ref[...]], axis=1)
    z = _gelu(jnp.dot(x, w1_ref[...], preferred_element_type=jnp.float32)
              + b1_ref[...])
    z = _gelu(jnp.dot(z, w2_ref[...], preferred_element_type=jnp.float32)
              + b2_ref[...])
    o_ref[...] = jnp.dot(z, w3_ref[...],
                         preferred_element_type=jnp.float32) + b3_ref[...]


def _ffn(agg, h, W1, b1, W2, b2, W3, b3):
    grid = _N // _BLK
    return pl.pallas_call(
        _ffn_body,
        grid=(grid,),
        in_specs=[pl.BlockSpec((2, 2, _BLK, 128), lambda i: (0, 0, i, 0)),
                  pl.BlockSpec((_BLK, _D), lambda i: (i, 0)),
                  pl.BlockSpec((2 * _D, _F), lambda i: (0, 0)),
                  pl.BlockSpec((1, _F), lambda i: (0, 0)),
                  pl.BlockSpec((_F, _F), lambda i: (0, 0)),
                  pl.BlockSpec((1, _F), lambda i: (0, 0)),
                  pl.BlockSpec((_F, _D), lambda i: (0, 0)),
                  pl.BlockSpec((1, _D), lambda i: (0, 0))],
        out_specs=pl.BlockSpec((_BLK, _D), lambda i: (i, 0)),
        out_shape=jax.ShapeDtypeStruct((_N, _D), jnp.float32),
    )(agg, h, W1, b1.reshape(1, _F), W2, b2.reshape(1, _F), W3,
      b3.reshape(1, _D))


def _sc_pass1(ngl, ngr, srcp, dstp, avec, zeros16):
    mesh = plsc.VectorSubcoreMesh(core_axis_name="c", subcore_axis_name="s")

    @functools.partial(
        pl.kernel, mesh=mesh,
        compiler_params=pltpu.CompilerParams(use_tc_tiling_on_sc=False,
                                             needs_layout_passes=False),
        out_type=[jax.ShapeDtypeStruct((_EEP, 16), jnp.float32),
                  jax.ShapeDtypeStruct((2, _NP, 16), jnp.float32)],
        scratch_types=[
            pltpu.VMEM((_B,), jnp.int32),
            pltpu.VMEM((_B,), jnp.int32),
            pltpu.VMEM((_B, _D), jnp.float32),
            pltpu.VMEM((_B, _D), jnp.float32),
            pltpu.VMEM((_B, 16), jnp.float32),
            pltpu.VMEM((32,), jnp.float32),
            pltpu.VMEM_SHARED((_NP, 16), jnp.float32),
            pltpu.SemaphoreType.DMA,
        ],
    )
    def k(ngl_h, ngr_h, src_h, dst_h, a_h, z_h, w_h, ssump_h,
          src_v, dst_v, gl_v, gr_v, w_v, a_v, sshare, sem):
        c = lax.axis_index("c")
        s = lax.axis_index("s")
        wid = s * 2 + c
        ebase = wid * _CH
        nrows = _NP // 16
        pltpu.sync_copy(a_h, a_v)
        pltpu.sync_copy(z_h, sshare.at[pl.ds(s * nrows, nrows)])

        lanes = lax.iota(jnp.int32, 16)
        zero16 = jnp.zeros((16,), jnp.float32)
        for g in range(_B // 16):
            rowi = g * 16 + lanes
            for col in range(16):
                plsc.store_scatter(
                    w_v, [rowi, jnp.full((16,), col, jnp.int32)], zero16)

        a0 = a_v[pl.ds(0, 16)]
        a1 = a_v[pl.ds(16, 16)]
        ascal = [a0[i] for i in range(16)] + [a1[i] for i in range(16)]
        plsc.subcore_barrier()

        def block(i, _):
            base = ebase + i * _B
            pltpu.sync_copy(src_h.at[pl.ds(base, _B)], src_v)
            pltpu.sync_copy(dst_h.at[pl.ds(base, _B)], dst_v)
            pltpu.async_copy(ngl_h.at[src_v], gl_v, sem).wait()
            pltpu.async_copy(ngr_h.at[dst_v], gr_v, sem).wait()
            for g in range(_B // 16):
                rowi = g * 16 + lanes
                eid = (base + g * 16) + lanes
                keep = eid < _EE
                for h in range(_H):
                    acc = jnp.zeros((16,), jnp.float32)
                    for d in range(_DH):
                        col = jnp.full((16,), h * _DH + d, jnp.int32)
                        glv = plsc.load_gather(gl_v, [rowi, col])
                        grv = plsc.load_gather(gr_v, [rowi, col])
                        sv = glv + grv
                        acc = acc + jnp.maximum(sv, 0.2 * sv) * ascal[d]
                    wv = jnp.where(keep, jnp.exp(acc), 0.0)
                    plsc.store_scatter(
                        w_v, [rowi, jnp.full((16,), h, jnp.int32)], wv)
            pltpu.sync_copy(w_v, w_h.at[pl.ds(base, _B)])
            pltpu.sync_copy(w_v, sshare.at[src_v], add=True)
            return 0
        lax.fori_loop(0, _NBLK, block, 0)
        plsc.subcore_barrier()
        pltpu.sync_copy(sshare.at[pl.ds(s * nrows, nrows)],
                        ssump_h.at[c, pl.ds(s * nrows, nrows)])

    return k(ngl, ngr, srcp, dstp, avec, zeros16)


def _sc_pass2(ngr2, srcp, dstp, w, ssum16, zeros626):
    mesh = plsc.VectorSubcoreMesh(core_axis_name="c", subcore_axis_name="s")

    @functools.partial(
        pl.kernel, mesh=mesh,
        compiler_params=pltpu.CompilerParams(use_tc_tiling_on_sc=False,
                                             needs_layout_passes=False),
        out_type=jax.ShapeDtypeStruct((2, 2, _NP, 128), jnp.float32),
        scratch_types=[
            pltpu.VMEM((_B,), jnp.int32),
            pltpu.VMEM((_B,), jnp.int32),
            pltpu.VMEM((_B,), jnp.int32),
            pltpu.VMEM((_B,), jnp.int32),
            pltpu.VMEM((_B, 128), jnp.float32),
            pltpu.VMEM((_B, 128), jnp.float32),
            pltpu.VMEM((_B, 128), jnp.float32),
            pltpu.VMEM((_B, 16), jnp.float32),
            pltpu.VMEM((_B, 16), jnp.float32),
            pltpu.VMEM((_B, 16), jnp.float32),
            pltpu.VMEM_SHARED((_NP, 128), jnp.float32),
            pltpu.SemaphoreType.DMA,
        ],
    )
    def k(ngr2_h, src_h, dst_h, w_h, ssum_h, z_h, agg_h,
          src_v, dst_v, dst2_v, rowid_v, gr_v, be_v, be2_v, wblk_v,
          ssb_v, ssb0_v, shared, sem):
        c = lax.axis_index("c")
        s = lax.axis_index("s")
        wid = s * 2 + c
        ebase = wid * _CH
        nrows = _NP // 16
        myrows = s * nrows
        lanes = lax.iota(jnp.int32, 16)

        for half in range(2):
            pltpu.sync_copy(z_h, shared.at[pl.ds(myrows, nrows)])
            plsc.subcore_barrier()

            def block(i, _):
                base = ebase + i * _B
                pltpu.sync_copy(src_h.at[pl.ds(base, _B)], src_v)
                pltpu.sync_copy(dst_h.at[pl.ds(base, _B)], dst_v)
                pltpu.sync_copy(w_h.at[pl.ds(base, _B)], wblk_v)
                pltpu.async_copy(ssum_h.at[src_v], ssb_v, sem).wait()
                for g in range(_B // 16):
                    dv = dst_v[pl.ds(g * 16, 16)]
                    dst2_v[pl.ds(g * 16, 16)] = dv * 2 + half
                pltpu.async_copy(ngr2_h.at[dst2_v], gr_v, sem).wait()
                for g in range(_B // 16):
                    rowi = g * 16 + lanes
                    for h4 in range(4):
                        h = half * 4 + h4
                        hcol = jnp.full((16,), h, jnp.int32)
                        wv = plsc.load_gather(wblk_v, [rowi, hcol])
                        ssv = plsc.load_gather(ssb_v, [rowi, hcol])
                        av = jnp.where(wv > 0.0, wv / ssv, 0.0)

                        def dbody(d, _):
                            col = jnp.full((16,), h4 * _DH, jnp.int32) + d
                            grv = plsc.load_gather(gr_v, [rowi, col])
                            plsc.store_scatter(be_v, [rowi, col], grv * av)
                            return 0
                        lax.fori_loop(0, _DH, dbody, 0)
                pltpu.sync_copy(be_v, shared.at[src_v], add=True)

                @pl.when(wid == 0)
                def _():
                    for g in range(_B // 16):
                        nid = (base + g * 16) + lanes
                        rowid_v[pl.ds(g * 16, 16)] = nid
                    pltpu.async_copy(ssum_h.at[rowid_v], ssb0_v, sem).wait()
                    for g in range(_B // 16):
                        rowi = g * 16 + lanes
                        nid = (base + g * 16) + lanes
                        ss0 = plsc.load_gather(
                            ssb0_v, [rowi, jnp.full((16,), 0, jnp.int32)])
                        mv = jnp.where((ss0 == 0.0) & (nid < _N), 1.0, 0.0)

                        def mbody(d, _):
                            col = jnp.full((16,), 0, jnp.int32) + d
                            bv = plsc.load_gather(be_v, [rowi, col])
                            plsc.store_scatter(be2_v, [rowi, col], bv * mv)
                            return 0
                        lax.fori_loop(0, 128, mbody, 0)
                    pltpu.sync_copy(be2_v, shared.at[rowid_v], add=True)
                return 0
            lax.fori_loop(0, _NBLK, block, 0)
            plsc.subcore_barrier()
            pltpu.sync_copy(shared.at[pl.ds(myrows, nrows)],
                            agg_h.at[c, half, pl.ds(myrows, nrows)])
            plsc.subcore_barrier()

    return k(ngr2, srcp, dstp, w, ssum16, zeros626)


def kernel(atoms, bonds, tetras, cistrans, emb, marker, Wl, Wr, a, W1, b1,
           W2, b2, W3, b3):
    idx = jnp.concatenate([bonds, bonds[:, ::-1]], axis=0).astype(jnp.int32)
    pad = jnp.zeros((_EEP - _EE, 2), jnp.int32)
    idxp = jnp.concatenate([idx, pad], axis=0)
    srcp = idxp[:, 0]
    dstp = idxp[:, 1]
    zeros626 = jnp.zeros((_NP // 16, 128), jnp.float32)
    zeros16 = jnp.zeros((_NP // 16, 16), jnp.float32)

    h = None
    for l in range(_L):
        if l == 0:
            Wlr = jnp.concatenate([Wl[0], Wr[0]], axis=1)
            h, ngl, ngr = _layer0(atoms, emb, Wlr)
        else:
            ngl, ngr = _matmul_ng(h, Wl[l], Wr[l])
        w, ssparts = _sc_pass1(ngl, ngr, srcp, dstp, a[l], zeros16)
        ssum16 = _reduce_ssum(ssparts.reshape(2, _NP * 16)).reshape(_NP, 16)
        ngr2 = ngr.reshape(2 * _N, 128)
        agg = _sc_pass2(ngr2, srcp, dstp, w, ssum16, zeros626)
        h = _ffn(agg, h, W1[l], b1[l], W2[l], b2[l], W3[l], b3[l])
    return h
